# baseline (speedup 1.0000x reference)
import functools

import jax
import jax.numpy as jnp
from jax import lax
from jax.experimental import pallas as pl
from jax.experimental.pallas import tpu as pltpu
from jax.experimental.pallas import tpu_sc as plsc

VOCAB = 100000
EMBED = 64
CTX = 20
AW = 128
AK = 80
VPAD = 100352
BV = 2048
NEG = -1e30
INFLIGHT = 20


def _sc_gather(t_table, idx_t, tgt_idx, tb_idx, bp, zeros_blk):
    n_idx = idx_t.shape[0]
    n_tgt = tgt_idx.shape[0]
    info = plsc.get_sparse_core_info()
    nw = info.num_cores * info.num_subcores
    tgt_per_w = n_tgt // nw
    chw = tgt_per_w
    ch_per_w = n_idx // (nw * chw)

    mesh = plsc.VectorSubcoreMesh(core_axis_name="c", subcore_axis_name="s")

    @functools.partial(
        pl.kernel,
        mesh=mesh,
        out_type=[
            jax.ShapeDtypeStruct((n_tgt, AW), jnp.float32),
            jax.ShapeDtypeStruct((n_tgt, AW), jnp.float32),
            jax.ShapeDtypeStruct((n_tgt, AW), jnp.float32),
        ],
        scratch_types=[
            pltpu.VMEM((ch_per_w * chw,), jnp.int32),
            pltpu.VMEM((chw, AW), jnp.float32),
            pltpu.VMEM((tgt_per_w,), jnp.int32),
            pltpu.VMEM((tgt_per_w,), jnp.int32),
            pltpu.VMEM((tgt_per_w, AW), jnp.float32),
            pltpu.VMEM((tgt_per_w, AW), jnp.float32),
            pltpu.SemaphoreType.DMA,
            pltpu.SemaphoreType.DMA,
            pltpu.SemaphoreType.DMA,
        ],
    )
    def k(t_hbm, idx_hbm, tgt_hbm, tbi_hbm, bp_hbm, z_hbm,
          s_out, tg_out, bg_out,
          idx_all, acc, tidx, tbidx, trows, brows, gs, ts, ws):
        wid = lax.axis_index("s") * info.num_cores + lax.axis_index("c")
        base = wid * tgt_per_w
        pltpu.sync_copy(
            idx_hbm.at[pl.ds(wid * ch_per_w * chw, ch_per_w * chw)], idx_all)
        pltpu.sync_copy(tgt_hbm.at[pl.ds(base, tgt_per_w)], tidx)
        pltpu.sync_copy(tbi_hbm.at[pl.ds(base, tgt_per_w)], tbidx)
        th = pltpu.async_copy(t_hbm.at[tidx], trows, ts)
        bh = pltpu.async_copy(bp_hbm.at[tbidx], brows, ts)
        pltpu.sync_copy(z_hbm, acc)
        gh = [None] * ch_per_w
        for j in range(ch_per_w):
            if j >= INFLIGHT:
                gh[j - INFLIGHT].wait()
            gh[j] = pltpu.async_copy(
                t_hbm.at[idx_all.at[pl.ds(j * chw, chw)]], acc, gs, add=True)
        for j in range(ch_per_w - INFLIGHT, ch_per_w):
            gh[j].wait()
        w0 = pltpu.async_copy(acc, s_out.at[pl.ds(base, tgt_per_w)], ws)
        th.wait()
        w1 = pltpu.async_copy(trows, tg_out.at[pl.ds(base, tgt_per_w)], ws)
        bh.wait()
        w2 = pltpu.async_copy(brows, bg_out.at[pl.ds(base, tgt_per_w)], ws)
        w0.wait()
        w1.wait()
        w2.wait()

    return k(t_table, idx_t, tgt_idx, tb_idx, bp, zeros_blk)


def _tc_body(s_ref, tg_ref, bg_ref, oh_ref, at_ref, out_ref,
             sbf_ref, tgt_ref, acc_ref):
    iv = pl.program_id(0)
    nv = pl.num_programs(0)
    nb_tot = s_ref.shape[0]

    @pl.when(iv == 0)
    def _init():
        s = s_ref[:, 0:EMBED]
        s_aug = jnp.concatenate(
            [s, jnp.ones((nb_tot, 1), jnp.float32),
             jnp.zeros((nb_tot, AK - EMBED - 1), jnp.float32)], axis=1)
        sbf_ref[...] = s_aug.astype(jnp.bfloat16)
        tgt_ref[...] = (
            jnp.sum(s * tg_ref[:, EMBED:AW], axis=1, keepdims=True)
            + jnp.sum(bg_ref[...] * oh_ref[...], axis=1, keepdims=True))
        acc_ref[...] = jnp.zeros_like(acc_ref)

    logits = lax.dot_general(
        sbf_ref[...], at_ref[...], (((1,), (0,)), ((), ())),
        preferred_element_type=jnp.float32)
    acc_ref[...] += jnp.sum(jnp.exp2(logits), axis=1, keepdims=True)

    @pl.when(iv == nv - 1)
    def _fin():
        nll = jnp.log(acc_ref[...]) - tgt_ref[...]
        out_ref[...] = jnp.full((1, 1), jnp.sum(nll) / nb_tot, jnp.float32)


def _tc_loss(s, tg, bg, oh, at_bf):
    b = s.shape[0]
    nv = VPAD // BV
    full = lambda i: (0, 0)
    out = pl.pallas_call(
        _tc_body,
        grid=(nv,),
        in_specs=[
            pl.BlockSpec((b, AW), full),
            pl.BlockSpec((b, AW), full),
            pl.BlockSpec((b, AW), full),
            pl.BlockSpec((b, AW), full),
            pl.BlockSpec((AK, BV), lambda i: (0, i)),
        ],
        out_specs=pl.BlockSpec((1, 1), full),
        out_shape=jax.ShapeDtypeStruct((1, 1), jnp.float32),
        scratch_shapes=[
            pltpu.VMEM((b, AK), jnp.bfloat16),
            pltpu.VMEM((b, 1), jnp.float32),
            pltpu.VMEM((b, 1), jnp.float32),
        ],
    )(s, tg, bg, oh, at_bf)
    return out[0, 0]


def kernel(inputs, target, emb_table, W, b):
    bsz, ctx = inputs.shape
    v, e = emb_table.shape
    nw = 32
    half = bsz // 2
    per_w = half // nw
    idx_t = [
        (inputs[h * half:(h + 1) * half].astype(jnp.int32)
         .reshape(nw, per_w, ctx).transpose(0, 2, 1).reshape(-1))
        for h in range(2)
    ]
    tgt_idx = target.astype(jnp.int32)
    tb_idx = tgt_idx // AW
    oh = jax.nn.one_hot(tgt_idx % AW, AW, dtype=jnp.float32)
    t_table = jnp.concatenate(
        [emb_table[:, None, :], W[:, None, :]], axis=1).reshape(v, 2 * e)
    bp = jnp.concatenate(
        [b, jnp.zeros((-v) % AW, jnp.float32)]).reshape(-1, AW)
    zeros_blk = jnp.zeros((per_w, AW), jnp.float32)
    halves = [
        _sc_gather(t_table, idx_t[h], tgt_idx[h * half:(h + 1) * half],
                   tb_idx[h * half:(h + 1) * half], bp, zeros_blk)
        for h in range(2)
    ]
    pad_cols = jnp.zeros((AK, VPAD - v), jnp.float32).at[e, :].set(NEG)
    at_t = jnp.concatenate(
        [W.T, b[None, :], jnp.zeros((AK - e - 1, v), jnp.float32)], axis=0)
    at_t = (jnp.concatenate([at_t, pad_cols], axis=1)
            * jnp.float32(1.4426950408889634)).astype(jnp.bfloat16)
    l0 = _tc_loss(halves[0][0], halves[0][1], halves[0][2],
                  oh[0:half], at_t)
    l1 = _tc_loss(halves[1][0], halves[1][1], halves[1][2],
                  oh[half:], at_t)
    return (l0 + l1) * 0.5


# speedup vs baseline: 1.0494x; 1.0494x over previous
import functools

import jax
import jax.numpy as jnp
from jax import lax
from jax.experimental import pallas as pl
from jax.experimental.pallas import tpu as pltpu
from jax.experimental.pallas import tpu_sc as plsc

VOCAB = 100000
EMBED = 64
CTX = 20
AW = 128
AK = 80
VPAD = 100352
BV = 2048
NEG = -1e30
INFLIGHT = 20


def _sc_gather(t_table, idx_t, tgt_idx, tb_idx, bp, zeros_blk):
    n_idx = idx_t.shape[0]
    n_tgt = tgt_idx.shape[0]
    info = plsc.get_sparse_core_info()
    nw = info.num_cores * info.num_subcores
    chw = 128
    ch_per_w = n_idx // (nw * chw)
    tgt_per_w = n_tgt // nw

    mesh = plsc.VectorSubcoreMesh(core_axis_name="c", subcore_axis_name="s")

    @functools.partial(
        pl.kernel,
        mesh=mesh,
        out_type=[
            jax.ShapeDtypeStruct((n_tgt, AW), jnp.float32),
            jax.ShapeDtypeStruct((n_tgt, AW), jnp.float32),
            jax.ShapeDtypeStruct((n_tgt, AW), jnp.float32),
        ],
        scratch_types=[
            pltpu.VMEM((ch_per_w * chw,), jnp.int32),
            pltpu.VMEM((chw, AW), jnp.float32),
            pltpu.VMEM((tgt_per_w,), jnp.int32),
            pltpu.VMEM((tgt_per_w,), jnp.int32),
            pltpu.VMEM((tgt_per_w, AW), jnp.float32),
            pltpu.VMEM((tgt_per_w, AW), jnp.float32),
            pltpu.SemaphoreType.DMA,
            pltpu.SemaphoreType.DMA,
            pltpu.SemaphoreType.DMA,
        ],
    )
    def k(t_hbm, idx_hbm, tgt_hbm, tbi_hbm, bp_hbm, z_hbm,
          s_out, tg_out, bg_out,
          idx_all, acc, tidx, tbidx, trows, brows, gs, ts, ws):
        wid = lax.axis_index("s") * info.num_cores + lax.axis_index("c")
        base = wid * tgt_per_w
        pltpu.sync_copy(
            idx_hbm.at[pl.ds(wid * ch_per_w * chw, ch_per_w * chw)], idx_all)
        pltpu.sync_copy(tgt_hbm.at[pl.ds(base, tgt_per_w)], tidx)
        pltpu.sync_copy(tbi_hbm.at[pl.ds(base, tgt_per_w)], tbidx)
        th = pltpu.async_copy(t_hbm.at[tidx], trows, ts)
        bh = pltpu.async_copy(bp_hbm.at[tbidx], brows, ts)
        pltpu.sync_copy(z_hbm, acc)
        gh = [None] * ch_per_w
        for j in range(ch_per_w):
            if j >= INFLIGHT:
                gh[j - INFLIGHT].wait()
            gh[j] = pltpu.async_copy(
                t_hbm.at[idx_all.at[pl.ds(j * chw, chw)]], acc, gs, add=True)
        for j in range(ch_per_w - INFLIGHT, ch_per_w):
            gh[j].wait()
        w0 = pltpu.async_copy(acc, s_out.at[pl.ds(base, tgt_per_w)], ws)
        th.wait()
        w1 = pltpu.async_copy(trows, tg_out.at[pl.ds(base, tgt_per_w)], ws)
        bh.wait()
        w2 = pltpu.async_copy(brows, bg_out.at[pl.ds(base, tgt_per_w)], ws)
        w0.wait()
        w1.wait()
        w2.wait()

    return k(t_table, idx_t, tgt_idx, tb_idx, bp, zeros_blk)


def _tc_body(s_ref, tg_ref, bg_ref, oh_ref, at_ref, out_ref,
             sbf_ref, tgt_ref, acc_ref):
    iv = pl.program_id(0)
    nv = pl.num_programs(0)
    nb_tot = s_ref.shape[0]

    @pl.when(iv == 0)
    def _init():
        s = s_ref[:, 0:EMBED]
        s_aug = jnp.concatenate(
            [s, jnp.ones((nb_tot, 1), jnp.float32),
             jnp.zeros((nb_tot, AK - EMBED - 1), jnp.float32)], axis=1)
        sbf_ref[...] = s_aug.astype(jnp.bfloat16)
        tgt_ref[...] = (
            jnp.sum(s * tg_ref[:, EMBED:AW], axis=1, keepdims=True)
            + jnp.sum(bg_ref[...] * oh_ref[...], axis=1, keepdims=True))
        acc_ref[...] = jnp.zeros_like(acc_ref)

    logits = lax.dot_general(
        sbf_ref[...], at_ref[...], (((1,), (0,)), ((), ())),
        preferred_element_type=jnp.float32)
    ex = jnp.exp2(logits)
    part = ex[:, 0:AW]
    for k in range(1, BV // AW):
        part = part + ex[:, k * AW:(k + 1) * AW]
    acc_ref[...] += part

    @pl.when(iv == nv - 1)
    def _fin():
        tot = jnp.sum(acc_ref[...], axis=1, keepdims=True)
        nll = jnp.log(tot) - tgt_ref[...]
        out_ref[...] = jnp.full((1, 1), jnp.sum(nll) / nb_tot, jnp.float32)


def _tc_loss(s, tg, bg, oh, at_bf):
    b = s.shape[0]
    nv = VPAD // BV
    full = lambda i: (0, 0)
    out = pl.pallas_call(
        _tc_body,
        grid=(nv,),
        in_specs=[
            pl.BlockSpec((b, AW), full),
            pl.BlockSpec((b, AW), full),
            pl.BlockSpec((b, AW), full),
            pl.BlockSpec((b, AW), full),
            pl.BlockSpec((AK, BV), lambda i: (0, i)),
        ],
        out_specs=pl.BlockSpec((1, 1), full),
        out_shape=jax.ShapeDtypeStruct((1, 1), jnp.float32),
        scratch_shapes=[
            pltpu.VMEM((b, AK), jnp.bfloat16),
            pltpu.VMEM((b, 1), jnp.float32),
            pltpu.VMEM((b, AW), jnp.float32),
        ],
    )(s, tg, bg, oh, at_bf)
    return out[0, 0]


def kernel(inputs, target, emb_table, W, b):
    bsz, ctx = inputs.shape
    v, e = emb_table.shape
    nw = 32
    per_w = bsz // nw
    idx_t = (inputs.astype(jnp.int32)
             .reshape(nw, per_w, ctx).transpose(0, 2, 1).reshape(-1))
    tgt_idx = target.astype(jnp.int32)
    tb_idx = tgt_idx // AW
    oh = jax.nn.one_hot(tgt_idx % AW, AW, dtype=jnp.float32)
    t_table = jnp.concatenate(
        [emb_table[:, None, :], W[:, None, :]], axis=1).reshape(v, 2 * e)
    bp = jnp.concatenate(
        [b, jnp.zeros((-v) % AW, jnp.float32)]).reshape(-1, AW)
    zeros_blk = jnp.zeros((per_w, AW), jnp.float32)
    s, tg, bg = _sc_gather(t_table, idx_t, tgt_idx, tb_idx, bp, zeros_blk)
    pad_cols = jnp.zeros((AK, VPAD - v), jnp.float32).at[e, :].set(NEG)
    at_t = jnp.concatenate(
        [W.T, b[None, :], jnp.zeros((AK - e - 1, v), jnp.float32)], axis=0)
    at_t = (jnp.concatenate([at_t, pad_cols], axis=1)
            * jnp.float32(1.4426950408889634)).astype(jnp.bfloat16)
    return _tc_loss(s, tg, bg, oh, at_t)
